# baseline (speedup 1.0000x reference)
import jax
import jax.numpy as jnp
from jax import lax
from jax.experimental import pallas as pl

B, K, H, W = 32, 17, 128, 128


def _fk(k):
    return jnp.where(k == 0, 0, jnp.where(k % 2 == 1, k + 1, k - 1))


def _antidiag():
    r = lax.broadcasted_iota(jnp.int32, (W, W), 0)
    c = lax.broadcasted_iota(jnp.int32, (W, W), 1)
    return jnp.where(r + c == W - 1, 1.0, 0.0).astype(jnp.float32)


def _flip4(x, bb):
    def body(in_ref, out_ref):
        j = _antidiag()
        for i in range(bb):
            out_ref[i, 0] = jax.lax.dot(
                in_ref[i, 0], j,
                preferred_element_type=jnp.float32)

    return pl.pallas_call(
        body,
        grid=(B // bb, K),
        in_specs=[pl.BlockSpec((bb, 1, H, W), lambda b, k: (b, _fk(k), 0, 0))],
        out_specs=pl.BlockSpec((bb, 1, H, W), lambda b, k: (b, k, 0, 0)),
        out_shape=jax.ShapeDtypeStruct((B, K, H, W), jnp.float32),
    )(x)


def _flip5(x, bb):
    def body(in_ref, out_ref):
        j = _antidiag()
        c = pl.program_id(2)
        sign = jnp.where(c == 0, -1.0, 1.0)
        for i in range(bb):
            out_ref[i, 0, 0] = sign * jax.lax.dot(
                in_ref[i, 0, 0], j,
                preferred_element_type=jnp.float32)

    return pl.pallas_call(
        body,
        grid=(B // bb, K, 2),
        in_specs=[pl.BlockSpec((bb, 1, 1, H, W),
                               lambda b, k, c: (b, _fk(k), c, 0, 0))],
        out_specs=pl.BlockSpec((bb, 1, 1, H, W),
                               lambda b, k, c: (b, k, c, 0, 0)),
        out_shape=jax.ShapeDtypeStruct((B, K, 2, H, W), jnp.float32),
    )(x)


def kernel(field_conf, field_reg, field_scale):
    return (_flip4(field_conf, 8), _flip5(field_reg, 8), _flip4(field_scale, 8))


# speedup vs baseline: 1.9119x; 1.9119x over previous
import functools

import jax
import jax.numpy as jnp
from jax import lax
from jax.experimental import pallas as pl
from jax.experimental.pallas import tpu as pltpu
from jax.experimental.pallas import tpu_sc as plsc

B, K, H, W = 32, 17, 128, 128
HW = H * W
_FI = tuple(0 if k == 0 else (k + 1 if k % 2 == 1 else k - 1) for k in range(K))

_CHUNKS_PER_ROW = W // 16


def _rev_plane(vin, vout, negate):
    def body(h, carry):
        base = h * W
        for j in range(_CHUNKS_PER_ROW):
            src = base + (_CHUNKS_PER_ROW - 1 - j) * 16
            v = lax.rev(vin[pl.ds(src, 16)], (0,))
            if negate:
                v = -v
            vout[pl.ds(base + j * 16, 16)] = v
        return carry

    lax.fori_loop(0, H, body, 0)


def _sc_flip(conf, reg):
    mesh = plsc.VectorSubcoreMesh(core_axis_name="c", subcore_axis_name="s")

    @functools.partial(
        pl.kernel,
        mesh=mesh,
        out_type=(
            jax.ShapeDtypeStruct((B, K, HW), jnp.float32),
            jax.ShapeDtypeStruct((B, K, 2, HW), jnp.float32),
        ),
        scratch_types=[
            pltpu.VMEM((HW,), jnp.float32),
            pltpu.VMEM((HW,), jnp.float32),
            pltpu.VMEM((HW,), jnp.float32),
            pltpu.VMEM((HW,), jnp.float32),
            pltpu.VMEM((HW,), jnp.float32),
            pltpu.VMEM((HW,), jnp.float32),
            pltpu.SemaphoreType.DMA,
            pltpu.SemaphoreType.DMA,
            pltpu.SemaphoreType.DMA,
            pltpu.SemaphoreType.DMA,
            pltpu.SemaphoreType.DMA,
            pltpu.SemaphoreType.DMA,
        ],
        compiler_params=pltpu.CompilerParams(use_tc_tiling_on_sc=False),
    )
    def k(conf_in, reg_in, conf_out, reg_out,
          bin0, bin1, bin2, bout0, bout1, bout2,
          isem0, isem1, isem2, osem0, osem1, osem2):
        w = lax.axis_index("s") * 2 + lax.axis_index("c")
        bins, bouts = (bin0, bin1, bin2), (bout0, bout1, bout2)
        isems, osems = (isem0, isem1, isem2), (osem0, osem1, osem2)

        planes = []
        for kk in range(K):
            planes.append((conf_in.at[w, _FI[kk]], conf_out.at[w, kk], False))
        for c in range(2):
            for kk in range(K):
                planes.append(
                    (reg_in.at[w, _FI[kk], c], reg_out.at[w, kk, c], c == 0))
        n = len(planes)

        d = 3
        copy_in = [None] * n
        copy_out = [None] * n
        for i in range(d):
            copy_in[i] = pltpu.async_copy(planes[i][0], bins[i], isems[i])
        for i in range(n):
            s = i % d
            copy_in[i].wait()
            if i >= d:
                copy_out[i - d].wait()
            _rev_plane(bins[s], bouts[s], planes[i][2])
            copy_out[i] = pltpu.async_copy(bouts[s], planes[i][1], osems[s])
            if i + d < n:
                copy_in[i + d] = pltpu.async_copy(
                    planes[i + d][0], bins[s], isems[s])
        for i in range(n - d, n):
            copy_out[i].wait()

    return k(conf, reg)


def _fk(k):
    return jnp.where(k == 0, 0, jnp.where(k % 2 == 1, k + 1, k - 1))


def _tc_flip(x, bb=8):
    def body(in_ref, out_ref):
        r = lax.broadcasted_iota(jnp.int32, (W, W), 0)
        c = lax.broadcasted_iota(jnp.int32, (W, W), 1)
        j = jnp.where(r + c == W - 1, 1.0, 0.0).astype(jnp.float32)
        for i in range(bb):
            out_ref[i, 0] = jax.lax.dot(
                in_ref[i, 0], j, preferred_element_type=jnp.float32)

    return pl.pallas_call(
        body,
        grid=(B // bb, K),
        in_specs=[pl.BlockSpec((bb, 1, H, W), lambda b, k: (b, _fk(k), 0, 0))],
        out_specs=pl.BlockSpec((bb, 1, H, W), lambda b, k: (b, k, 0, 0)),
        out_shape=jax.ShapeDtypeStruct((B, K, H, W), jnp.float32),
    )(x)


def kernel(field_conf, field_reg, field_scale):
    conf = field_conf.reshape(B, K, HW)
    reg = field_reg.reshape(B, K, 2, HW)
    oc, orr = _sc_flip(conf, reg)
    osc = _tc_flip(field_scale)
    return (
        oc.reshape(B, K, H, W),
        orr.reshape(B, K, 2, H, W),
        osc,
    )
